# baseline (speedup 1.0000x reference)
import jax
import jax.numpy as jnp
import numpy as np
from jax import lax
from jax.experimental import pallas as pl
from jax.experimental.pallas import tpu as pltpu
from jax.experimental.pallas import tpu_sc as plsc

N = 10000
E = 160000
T = 4
F = 64
HID = 256
EDGE_DIM = 16

_DEG_HIST = np.array([0,1,2,5,11,23,44,79,135,216,324,457,605,753,880,966,997,966,880,753,605,457,324,216,135,79,44,23,11,5,2,1,0], dtype=np.float64)
_bins_ = np.arange(_DEG_HIST.shape[0], dtype=np.float64)
_AVG_LOG = float((np.log(_bins_ + 1.0) * _DEG_HIST).sum() / float(_DEG_HIST.sum()))

NP = 10240
NB = 1024
EB2 = 1280
EB = 3200

_NC = 2
_NS = 16
_NW = _NC * _NS
EPW = E // _NW
IC = 1000
GC = 40
CPW = 2
ROUNDS = HID // (_NW * CPW)
BIG = 3.0e38


def _fold_kernel(edge_W_ref, edge_b_ref, pre_W1_ref, pre_b1_ref, cw_ref, cbias_ref):
    for t in range(T):
        w1c = pre_W1_ref[t, 128:192, :]
        cw_ref[:, t * F:(t + 1) * F] = jnp.dot(edge_W_ref[...], w1c,
                                               preferred_element_type=jnp.float32)
        cb = jnp.dot(edge_b_ref[...], w1c, preferred_element_type=jnp.float32)
        cbias_ref[0, t * F:(t + 1) * F] = cb + pre_b1_ref[t, :]


def _tables_kernel(x_ref, pre_W1_ref, a_ref, b_ref):
    for t in range(T):
        xt = x_ref[:, t * F:(t + 1) * F]
        a_ref[:, t * F:(t + 1) * F] = jnp.dot(xt, pre_W1_ref[t, 0:F, :],
                                              preferred_element_type=jnp.float32)
        b_ref[:, t * F:(t + 1) * F] = jnp.dot(xt, pre_W1_ref[t, F:2 * F, :],
                                              preferred_element_type=jnp.float32)


def _sc_gather_body(a_hbm, b_hbm, dst_hbm, src_hbm, ga_hbm, gb_hbm,
                    idx_d, idx_s, rows_a, rows_b, sema, semb):
    wid = lax.axis_index("s") * _NC + lax.axis_index("c")
    base = wid * EPW

    def outer(jj, _):
        off = base + jj * IC
        pltpu.sync_copy(dst_hbm.at[pl.ds(off, IC)], idx_d)
        pltpu.sync_copy(src_hbm.at[pl.ds(off, IC)], idx_s)

        def inner(j, _):
            ca = pltpu.async_copy(a_hbm.at[idx_d.at[pl.ds(j * GC, GC)]],
                                  rows_a, sema)
            cb = pltpu.async_copy(b_hbm.at[idx_s.at[pl.ds(j * GC, GC)]],
                                  rows_b, semb)
            ca.wait()
            cb.wait()
            pltpu.sync_copy(rows_a, ga_hbm.at[pl.ds(off + j * GC, GC)])
            pltpu.sync_copy(rows_b, gb_hbm.at[pl.ds(off + j * GC, GC)])
            return 0

        lax.fori_loop(0, IC // GC, inner, 0)
        return 0

    lax.fori_loop(0, EPW // IC, outer, 0)


def _edge_kernel(ga_ref, gb_ref, ea_ref, cw_ref, cbias_ref, w2_ref, b2_ref, mT_ref):
    pre = (ga_ref[...] + gb_ref[...]
           + jnp.dot(ea_ref[...], cw_ref[...], preferred_element_type=jnp.float32)
           + cbias_ref[...])
    for t in range(T):
        mt = jnp.maximum(pre[:, t * F:(t + 1) * F], 0.0)
        mT_ref[t * F:(t + 1) * F, :] = (
            lax.dot_general(w2_ref[t], mt, (((0,), (1,)), ((), ())),
                            preferred_element_type=jnp.float32)
            + b2_ref[t, :][:, None])


def _vec_fill(ref, n, valvec):
    def body(p, _):
        ref[pl.ds(p * 16, 16)] = valvec
        return 0
    lax.fori_loop(0, n // 16, body, 0)


def _scatter_minmax(acc, idx, val, is_min):
    def cond(go):
        return go

    def body(_):
        old = plsc.load_gather(acc, [idx])
        need = (val < old) if is_min else (val > old)
        plsc.store_scatter(acc, [idx], val, mask=need)
        return jnp.any(need)

    lax.while_loop(cond, body, jnp.bool_(True))


def _sc_segment_body(mflat_hbm, dst_hbm, sflat_hbm, sqflat_hbm, mnflat_hbm,
                     mxflat_hbm, cnt_hbm,
                     acc_s, acc_q, acc_mn, acc_mx, acc_c, mbuf, dbuf):
    wid = lax.axis_index("s") * _NC + lax.axis_index("c")
    zero16 = jnp.zeros((16,), jnp.float32)
    big16 = jnp.full((16,), BIG, jnp.float32)
    one16 = jnp.ones((16,), jnp.float32)

    for r in range(ROUNDS):
        c0 = r * (_NW * CPW) + wid * CPW
        _vec_fill(acc_s, CPW * NP, zero16)
        _vec_fill(acc_q, CPW * NP, zero16)
        _vec_fill(acc_mn, CPW * NP, big16)
        _vec_fill(acc_mx, CPW * NP, -big16)
        if r == 0:
            @pl.when(wid == 0)
            def _():
                def cbody(p, _):
                    acc_c[pl.ds(p * 16, 16)] = zero16
                    return 0
                lax.fori_loop(0, NP // 16, cbody, 0)

        def chunk(ch, _):
            off = ch * EB
            pltpu.sync_copy(dst_hbm.at[pl.ds(off, EB)], dbuf)
            for k in range(CPW):
                pltpu.sync_copy(mflat_hbm.at[pl.ds((c0 + k) * E + off, EB)],
                                mbuf.at[pl.ds(k * EB, EB)])

            def one_vec(v):
                idx = dbuf[pl.ds(v * 16, 16)]
                _, lastm = plsc.scan_count(idx)
                nodup = jnp.all(lastm)
                vals = [mbuf[pl.ds(k * EB + v * 16, 16)] for k in range(CPW)]
                idxs = [idx + jnp.int32(k * NP) for k in range(CPW)]
                for k in range(CPW):
                    plsc.addupdate_scatter(acc_s, [idxs[k]], vals[k])
                    plsc.addupdate_scatter(acc_q, [idxs[k]], vals[k] * vals[k])

                @pl.when(nodup)
                def _():
                    for k in range(CPW):
                        oldn = plsc.load_gather(acc_mn, [idxs[k]])
                        plsc.store_scatter(acc_mn, [idxs[k]],
                                           jnp.minimum(oldn, vals[k]))
                        oldx = plsc.load_gather(acc_mx, [idxs[k]])
                        plsc.store_scatter(acc_mx, [idxs[k]],
                                           jnp.maximum(oldx, vals[k]))

                @pl.when(jnp.logical_not(nodup))
                def _():
                    for k in range(CPW):
                        _scatter_minmax(acc_mn, idxs[k], vals[k], True)
                        _scatter_minmax(acc_mx, idxs[k], vals[k], False)
                if r == 0:
                    @pl.when(wid == 0)
                    def _():
                        plsc.addupdate_scatter(acc_c, [idx], one16)

            def vec(v, _):
                one_vec(2 * v)
                one_vec(2 * v + 1)
                return 0

            lax.fori_loop(0, EB // 32, vec, 0)
            return 0

        lax.fori_loop(0, E // EB, chunk, 0)

        for k in range(CPW):
            pltpu.sync_copy(acc_s.at[pl.ds(k * NP, NP)],
                            sflat_hbm.at[pl.ds((c0 + k) * NP, NP)])
            pltpu.sync_copy(acc_q.at[pl.ds(k * NP, NP)],
                            sqflat_hbm.at[pl.ds((c0 + k) * NP, NP)])
            pltpu.sync_copy(acc_mn.at[pl.ds(k * NP, NP)],
                            mnflat_hbm.at[pl.ds((c0 + k) * NP, NP)])
            pltpu.sync_copy(acc_mx.at[pl.ds(k * NP, NP)],
                            mxflat_hbm.at[pl.ds((c0 + k) * NP, NP)])
        if r == 0:
            @pl.when(wid == 0)
            def _():
                pltpu.sync_copy(acc_c, cnt_hbm)


def _post_kernel(x_ref, sT_ref, mnT_ref, mxT_ref, sqT_ref, cnt_ref,
                 pw1_ref, pb1_ref, pw2_ref, pb2_ref, lw_ref, lb_ref,
                 lg_ref, lbeta_ref, out_ref):
    cnt = cnt_ref[...]
    deg = jnp.maximum(cnt, 1.0)
    inv_deg = 1.0 / deg
    has = (cnt > 0.0).astype(jnp.float32)
    lg1d = jnp.log(deg + 1.0)
    amp = lg1d * (1.0 / _AVG_LOG)
    att = _AVG_LOG / lg1d

    s = sT_ref[...]
    mean = s * inv_deg
    mn = mnT_ref[...] * has
    mx = mxT_ref[...] * has
    msq = sqT_ref[...] * inv_deg
    var = jnp.maximum(msq - mean * mean, 0.0)
    std = jnp.sqrt(var + 1e-5)

    dn = (((0,), (0,)), ((), ()))
    o_parts = []
    for t in range(T):
        sl = slice(t * F, (t + 1) * F)
        aggT = jnp.concatenate([s[sl], mean[sl], mn[sl], mx[sl], std[sl]],
                               axis=0)
        w = pw1_ref[t]
        otT = lax.dot_general(w[0:F], x_ref[:, sl], (((0,), (1,)), ((), ())),
                              preferred_element_type=jnp.float32)
        otT = otT + lax.dot_general(w[F:F + 5 * F], aggT, dn,
                                    preferred_element_type=jnp.float32)
        otT = otT + amp * lax.dot_general(w[F + 5 * F:F + 10 * F], aggT, dn,
                                          preferred_element_type=jnp.float32)
        otT = otT + att * lax.dot_general(w[F + 10 * F:F + 15 * F], aggT, dn,
                                          preferred_element_type=jnp.float32)
        otT = jnp.maximum(otT + pb1_ref[t, :][:, None], 0.0)
        otT = (lax.dot_general(pw2_ref[t], otT, dn,
                               preferred_element_type=jnp.float32)
               + pb2_ref[t, :][:, None])
        o_parts.append(otT)
    oT = jnp.concatenate(o_parts, axis=0)

    o = lax.dot_general(oT, lw_ref[...], dn,
                        preferred_element_type=jnp.float32) + lb_ref[0, :][None, :]
    mu = jnp.mean(o, axis=1, keepdims=True)
    vv = jnp.mean((o - mu) ** 2, axis=1, keepdims=True)
    ln = (o - mu) * lax.rsqrt(vv + 1e-5) * lg_ref[0, :][None, :] + lbeta_ref[0, :][None, :]
    out_ref[...] = x_ref[...] + jnp.maximum(ln, 0.0)


def kernel(x, prot_edge_index, prot_edge_attr, edge_W, edge_b, pre_W1, pre_b1,
           pre_W2, pre_b2, post_W1, post_b1, post_W2, post_b2, lin_W, lin_b,
           ln_g, ln_b):
    src = prot_edge_index[0]
    dst = prot_edge_index[1]

    cw, cbias = pl.pallas_call(
        _fold_kernel,
        out_shape=[jax.ShapeDtypeStruct((EDGE_DIM, HID), jnp.float32),
                   jax.ShapeDtypeStruct((1, HID), jnp.float32)],
    )(edge_W, edge_b, pre_W1, pre_b1)

    a_tab, b_tab = pl.pallas_call(
        _tables_kernel,
        grid=(N // 1000,),
        in_specs=[pl.BlockSpec((1000, HID), lambda i: (i, 0)),
                  pl.BlockSpec((T, 3 * F, F), lambda i: (0, 0, 0))],
        out_specs=[pl.BlockSpec((1000, HID), lambda i: (i, 0)),
                   pl.BlockSpec((1000, HID), lambda i: (i, 0))],
        out_shape=[jax.ShapeDtypeStruct((N, HID), jnp.float32),
                   jax.ShapeDtypeStruct((N, HID), jnp.float32)],
    )(x, pre_W1)

    mesh = plsc.VectorSubcoreMesh(core_axis_name="c", subcore_axis_name="s")
    ga, gb = pl.kernel(
        _sc_gather_body,
        out_type=[jax.ShapeDtypeStruct((E, HID), jnp.float32),
                  jax.ShapeDtypeStruct((E, HID), jnp.float32)],
        mesh=mesh,
        scratch_types=[pltpu.VMEM((IC,), jnp.int32),
                       pltpu.VMEM((IC,), jnp.int32),
                       pltpu.VMEM((GC, HID), jnp.float32),
                       pltpu.VMEM((GC, HID), jnp.float32),
                       pltpu.SemaphoreType.DMA,
                       pltpu.SemaphoreType.DMA],
    )(a_tab, b_tab, dst, src)

    mT = pl.pallas_call(
        _edge_kernel,
        grid=(E // EB2,),
        in_specs=[pl.BlockSpec((EB2, HID), lambda i: (i, 0)),
                  pl.BlockSpec((EB2, HID), lambda i: (i, 0)),
                  pl.BlockSpec((EB2, EDGE_DIM), lambda i: (i, 0)),
                  pl.BlockSpec((EDGE_DIM, HID), lambda i: (0, 0)),
                  pl.BlockSpec((1, HID), lambda i: (0, 0)),
                  pl.BlockSpec((T, F, F), lambda i: (0, 0, 0)),
                  pl.BlockSpec((T, F), lambda i: (0, 0))],
        out_specs=pl.BlockSpec((HID, EB2), lambda i: (0, i)),
        out_shape=jax.ShapeDtypeStruct((HID, E), jnp.float32),
    )(ga, gb, prot_edge_attr, cw, cbias, pre_W2, pre_b2)

    sT, sqT, mnT, mxT, cnt = pl.kernel(
        _sc_segment_body,
        out_type=[jax.ShapeDtypeStruct((HID * NP,), jnp.float32),
                  jax.ShapeDtypeStruct((HID * NP,), jnp.float32),
                  jax.ShapeDtypeStruct((HID * NP,), jnp.float32),
                  jax.ShapeDtypeStruct((HID * NP,), jnp.float32),
                  jax.ShapeDtypeStruct((NP,), jnp.float32)],
        mesh=plsc.VectorSubcoreMesh(core_axis_name="c", subcore_axis_name="s"),
        scratch_types=[pltpu.VMEM((CPW * NP,), jnp.float32),
                       pltpu.VMEM((CPW * NP,), jnp.float32),
                       pltpu.VMEM((CPW * NP,), jnp.float32),
                       pltpu.VMEM((CPW * NP,), jnp.float32),
                       pltpu.VMEM((NP,), jnp.float32),
                       pltpu.VMEM((CPW * EB,), jnp.float32),
                       pltpu.VMEM((EB,), jnp.int32)],
        compiler_params=pltpu.CompilerParams(needs_layout_passes=False),
    )(mT.reshape(HID * E), dst)
    sT = sT.reshape(HID, NP)
    sqT = sqT.reshape(HID, NP)
    mnT = mnT.reshape(HID, NP)
    mxT = mxT.reshape(HID, NP)

    xp = jnp.concatenate([x, jnp.zeros((NP - N, HID), jnp.float32)], axis=0)
    out = pl.pallas_call(
        _post_kernel,
        grid=(NP // NB,),
        in_specs=[pl.BlockSpec((NB, HID), lambda i: (i, 0)),
                  pl.BlockSpec((HID, NB), lambda i: (0, i)),
                  pl.BlockSpec((HID, NB), lambda i: (0, i)),
                  pl.BlockSpec((HID, NB), lambda i: (0, i)),
                  pl.BlockSpec((HID, NB), lambda i: (0, i)),
                  pl.BlockSpec((1, NB), lambda i: (0, i)),
                  pl.BlockSpec((T, 16 * F, F), lambda i: (0, 0, 0)),
                  pl.BlockSpec((T, F), lambda i: (0, 0)),
                  pl.BlockSpec((T, F, F), lambda i: (0, 0, 0)),
                  pl.BlockSpec((T, F), lambda i: (0, 0)),
                  pl.BlockSpec((HID, HID), lambda i: (0, 0)),
                  pl.BlockSpec((1, HID), lambda i: (0, 0)),
                  pl.BlockSpec((1, HID), lambda i: (0, 0)),
                  pl.BlockSpec((1, HID), lambda i: (0, 0))],
        out_specs=pl.BlockSpec((NB, HID), lambda i: (i, 0)),
        out_shape=jax.ShapeDtypeStruct((NP, HID), jnp.float32),
    )(xp, sT, mnT, mxT, sqT, cnt.reshape(1, NP), post_W1, post_b1, post_W2,
      post_b2, lin_W, lin_b.reshape(1, HID), ln_g.reshape(1, HID),
      ln_b.reshape(1, HID))
    return out[:N]


# speedup vs baseline: 1.0812x; 1.0812x over previous
import jax
import jax.numpy as jnp
import numpy as np
from jax import lax
from jax.experimental import pallas as pl
from jax.experimental.pallas import tpu as pltpu
from jax.experimental.pallas import tpu_sc as plsc

N = 10000
E = 160000
T = 4
F = 64
HID = 256
EDGE_DIM = 16

_DEG_HIST = np.array([0,1,2,5,11,23,44,79,135,216,324,457,605,753,880,966,997,966,880,753,605,457,324,216,135,79,44,23,11,5,2,1,0], dtype=np.float64)
_bins_ = np.arange(_DEG_HIST.shape[0], dtype=np.float64)
_AVG_LOG = float((np.log(_bins_ + 1.0) * _DEG_HIST).sum() / float(_DEG_HIST.sum()))

NP = 10240
NB = 1024
EB2 = 1280
EB = 6400

_NC = 2
_NS = 16
_NW = _NC * _NS
EPW = E // _NW
IC = 1000
GC = 40
CPW = 2
ROUNDS = HID // (_NW * CPW)
BIG = 3.0e38


def _fold_kernel(edge_W_ref, edge_b_ref, pre_W1_ref, pre_b1_ref, cw_ref, cbias_ref):
    for t in range(T):
        w1c = pre_W1_ref[t, 128:192, :]
        cw_ref[:, t * F:(t + 1) * F] = jnp.dot(edge_W_ref[...], w1c,
                                               preferred_element_type=jnp.float32)
        cb = jnp.dot(edge_b_ref[...], w1c, preferred_element_type=jnp.float32)
        cbias_ref[0, t * F:(t + 1) * F] = cb + pre_b1_ref[t, :]


def _tables_kernel(x_ref, pre_W1_ref, a_ref, b_ref):
    for t in range(T):
        xt = x_ref[:, t * F:(t + 1) * F]
        a_ref[:, t * F:(t + 1) * F] = jnp.dot(xt, pre_W1_ref[t, 0:F, :],
                                              preferred_element_type=jnp.float32)
        b_ref[:, t * F:(t + 1) * F] = jnp.dot(xt, pre_W1_ref[t, F:2 * F, :],
                                              preferred_element_type=jnp.float32)


def _sc_gather_body(a_hbm, b_hbm, dst_hbm, src_hbm, ga_hbm, gb_hbm,
                    idx_d, idx_s, rows_a, rows_b, sema, semb):
    wid = lax.axis_index("s") * _NC + lax.axis_index("c")
    base = wid * EPW

    def outer(jj, _):
        off = base + jj * IC
        pltpu.sync_copy(dst_hbm.at[pl.ds(off, IC)], idx_d)
        pltpu.sync_copy(src_hbm.at[pl.ds(off, IC)], idx_s)

        def inner(j, _):
            ca = pltpu.async_copy(a_hbm.at[idx_d.at[pl.ds(j * GC, GC)]],
                                  rows_a, sema)
            cb = pltpu.async_copy(b_hbm.at[idx_s.at[pl.ds(j * GC, GC)]],
                                  rows_b, semb)
            ca.wait()
            cb.wait()
            pltpu.sync_copy(rows_a, ga_hbm.at[pl.ds(off + j * GC, GC)])
            pltpu.sync_copy(rows_b, gb_hbm.at[pl.ds(off + j * GC, GC)])
            return 0

        lax.fori_loop(0, IC // GC, inner, 0)
        return 0

    lax.fori_loop(0, EPW // IC, outer, 0)


def _edge_kernel(ga_ref, gb_ref, ea_ref, cw_ref, cbias_ref, w2_ref, b2_ref, mT_ref):
    pre = (ga_ref[...] + gb_ref[...]
           + jnp.dot(ea_ref[...], cw_ref[...], preferred_element_type=jnp.float32)
           + cbias_ref[...])
    for t in range(T):
        mt = jnp.maximum(pre[:, t * F:(t + 1) * F], 0.0)
        mT_ref[t * F:(t + 1) * F, :] = (
            lax.dot_general(w2_ref[t], mt, (((0,), (1,)), ((), ())),
                            preferred_element_type=jnp.float32)
            + b2_ref[t, :][:, None])


def _vec_fill(ref, n, valvec):
    def body(p, _):
        ref[pl.ds(p * 16, 16)] = valvec
        return 0
    lax.fori_loop(0, n // 16, body, 0)


def _scatter_minmax(acc, idx, val, is_min):
    def cond(go):
        return go

    def body(_):
        old = plsc.load_gather(acc, [idx])
        need = (val < old) if is_min else (val > old)
        plsc.store_scatter(acc, [idx], val, mask=need)
        return jnp.any(need)

    lax.while_loop(cond, body, jnp.bool_(True))


def _sc_segment_body(mflat_hbm, dst_hbm, sflat_hbm, sqflat_hbm, mnflat_hbm,
                     mxflat_hbm, cnt_hbm,
                     acc_s, acc_q, acc_mn, acc_mx, acc_c, mbuf, dbuf, tmpi):
    wid = lax.axis_index("s") * _NC + lax.axis_index("c")
    zero16 = jnp.zeros((16,), jnp.float32)
    big16 = jnp.full((16,), BIG, jnp.float32)
    one16 = jnp.ones((16,), jnp.float32)

    for r in range(ROUNDS):
        c0 = r * (_NW * CPW) + wid * CPW
        _vec_fill(acc_s, CPW * NP, zero16)
        _vec_fill(acc_q, CPW * NP, zero16)
        _vec_fill(acc_mn, CPW * NP, big16)
        _vec_fill(acc_mx, CPW * NP, -big16)
        if r == 0:
            @pl.when(wid == 0)
            def _():
                def cbody(p, _):
                    acc_c[pl.ds(p * 16, 16)] = zero16
                    return 0
                lax.fori_loop(0, NP // 16, cbody, 0)

        def chunk(ch, _):
            off = ch * EB
            pltpu.sync_copy(dst_hbm.at[pl.ds(off, EB)], dbuf)
            for k in range(CPW):
                pltpu.sync_copy(mflat_hbm.at[pl.ds((c0 + k) * E + off, EB)],
                                mbuf.at[pl.ds(k * EB, EB)])

            lane = lax.iota(jnp.int32, 16)

            def one_vec(v):
                idx = dbuf[pl.ds(v * 16, 16)]
                plsc.store_scatter(tmpi, [idx], lane)
                back = plsc.load_gather(tmpi, [idx])
                nodup = jnp.all(back == lane)
                vals = [mbuf[pl.ds(k * EB + v * 16, 16)] for k in range(CPW)]
                idxs = [idx + jnp.int32(k * NP) for k in range(CPW)]
                for k in range(CPW):
                    plsc.addupdate_scatter(acc_s, [idxs[k]], vals[k])
                    plsc.addupdate_scatter(acc_q, [idxs[k]], vals[k] * vals[k])

                @pl.when(nodup)
                def _():
                    for k in range(CPW):
                        oldn = plsc.load_gather(acc_mn, [idxs[k]])
                        plsc.store_scatter(acc_mn, [idxs[k]],
                                           jnp.minimum(oldn, vals[k]))
                        oldx = plsc.load_gather(acc_mx, [idxs[k]])
                        plsc.store_scatter(acc_mx, [idxs[k]],
                                           jnp.maximum(oldx, vals[k]))

                @pl.when(jnp.logical_not(nodup))
                def _():
                    for k in range(CPW):
                        _scatter_minmax(acc_mn, idxs[k], vals[k], True)
                        _scatter_minmax(acc_mx, idxs[k], vals[k], False)
                if r == 0:
                    @pl.when(wid == 0)
                    def _():
                        plsc.addupdate_scatter(acc_c, [idx], one16)

            def vec(v, _):
                one_vec(2 * v)
                one_vec(2 * v + 1)
                return 0

            lax.fori_loop(0, EB // 32, vec, 0)
            return 0

        lax.fori_loop(0, E // EB, chunk, 0)

        for k in range(CPW):
            pltpu.sync_copy(acc_s.at[pl.ds(k * NP, NP)],
                            sflat_hbm.at[pl.ds((c0 + k) * NP, NP)])
            pltpu.sync_copy(acc_q.at[pl.ds(k * NP, NP)],
                            sqflat_hbm.at[pl.ds((c0 + k) * NP, NP)])
            pltpu.sync_copy(acc_mn.at[pl.ds(k * NP, NP)],
                            mnflat_hbm.at[pl.ds((c0 + k) * NP, NP)])
            pltpu.sync_copy(acc_mx.at[pl.ds(k * NP, NP)],
                            mxflat_hbm.at[pl.ds((c0 + k) * NP, NP)])
        if r == 0:
            @pl.when(wid == 0)
            def _():
                pltpu.sync_copy(acc_c, cnt_hbm)


def _post_kernel(x_ref, sT_ref, mnT_ref, mxT_ref, sqT_ref, cnt_ref,
                 pw1_ref, pb1_ref, pw2_ref, pb2_ref, lw_ref, lb_ref,
                 lg_ref, lbeta_ref, out_ref):
    cnt = cnt_ref[...]
    deg = jnp.maximum(cnt, 1.0)
    inv_deg = 1.0 / deg
    has = (cnt > 0.0).astype(jnp.float32)
    lg1d = jnp.log(deg + 1.0)
    amp = lg1d * (1.0 / _AVG_LOG)
    att = _AVG_LOG / lg1d

    s = sT_ref[...]
    mean = s * inv_deg
    mn = mnT_ref[...] * has
    mx = mxT_ref[...] * has
    msq = sqT_ref[...] * inv_deg
    var = jnp.maximum(msq - mean * mean, 0.0)
    std = jnp.sqrt(var + 1e-5)

    dn = (((0,), (0,)), ((), ()))
    o_parts = []
    for t in range(T):
        sl = slice(t * F, (t + 1) * F)
        aggT = jnp.concatenate([s[sl], mean[sl], mn[sl], mx[sl], std[sl]],
                               axis=0)
        w = pw1_ref[t]
        otT = lax.dot_general(w[0:F], x_ref[:, sl], (((0,), (1,)), ((), ())),
                              preferred_element_type=jnp.float32)
        otT = otT + lax.dot_general(w[F:F + 5 * F], aggT, dn,
                                    preferred_element_type=jnp.float32)
        otT = otT + amp * lax.dot_general(w[F + 5 * F:F + 10 * F], aggT, dn,
                                          preferred_element_type=jnp.float32)
        otT = otT + att * lax.dot_general(w[F + 10 * F:F + 15 * F], aggT, dn,
                                          preferred_element_type=jnp.float32)
        otT = jnp.maximum(otT + pb1_ref[t, :][:, None], 0.0)
        otT = (lax.dot_general(pw2_ref[t], otT, dn,
                               preferred_element_type=jnp.float32)
               + pb2_ref[t, :][:, None])
        o_parts.append(otT)
    oT = jnp.concatenate(o_parts, axis=0)

    o = lax.dot_general(oT, lw_ref[...], dn,
                        preferred_element_type=jnp.float32) + lb_ref[0, :][None, :]
    mu = jnp.mean(o, axis=1, keepdims=True)
    vv = jnp.mean((o - mu) ** 2, axis=1, keepdims=True)
    ln = (o - mu) * lax.rsqrt(vv + 1e-5) * lg_ref[0, :][None, :] + lbeta_ref[0, :][None, :]
    out_ref[...] = x_ref[...] + jnp.maximum(ln, 0.0)


def kernel(x, prot_edge_index, prot_edge_attr, edge_W, edge_b, pre_W1, pre_b1,
           pre_W2, pre_b2, post_W1, post_b1, post_W2, post_b2, lin_W, lin_b,
           ln_g, ln_b):
    src = prot_edge_index[0]
    dst = prot_edge_index[1]

    cw, cbias = pl.pallas_call(
        _fold_kernel,
        out_shape=[jax.ShapeDtypeStruct((EDGE_DIM, HID), jnp.float32),
                   jax.ShapeDtypeStruct((1, HID), jnp.float32)],
    )(edge_W, edge_b, pre_W1, pre_b1)

    a_tab, b_tab = pl.pallas_call(
        _tables_kernel,
        grid=(N // 1000,),
        in_specs=[pl.BlockSpec((1000, HID), lambda i: (i, 0)),
                  pl.BlockSpec((T, 3 * F, F), lambda i: (0, 0, 0))],
        out_specs=[pl.BlockSpec((1000, HID), lambda i: (i, 0)),
                   pl.BlockSpec((1000, HID), lambda i: (i, 0))],
        out_shape=[jax.ShapeDtypeStruct((N, HID), jnp.float32),
                   jax.ShapeDtypeStruct((N, HID), jnp.float32)],
    )(x, pre_W1)

    mesh = plsc.VectorSubcoreMesh(core_axis_name="c", subcore_axis_name="s")
    ga, gb = pl.kernel(
        _sc_gather_body,
        out_type=[jax.ShapeDtypeStruct((E, HID), jnp.float32),
                  jax.ShapeDtypeStruct((E, HID), jnp.float32)],
        mesh=mesh,
        scratch_types=[pltpu.VMEM((IC,), jnp.int32),
                       pltpu.VMEM((IC,), jnp.int32),
                       pltpu.VMEM((GC, HID), jnp.float32),
                       pltpu.VMEM((GC, HID), jnp.float32),
                       pltpu.SemaphoreType.DMA,
                       pltpu.SemaphoreType.DMA],
    )(a_tab, b_tab, dst, src)

    mT = pl.pallas_call(
        _edge_kernel,
        grid=(E // EB2,),
        in_specs=[pl.BlockSpec((EB2, HID), lambda i: (i, 0)),
                  pl.BlockSpec((EB2, HID), lambda i: (i, 0)),
                  pl.BlockSpec((EB2, EDGE_DIM), lambda i: (i, 0)),
                  pl.BlockSpec((EDGE_DIM, HID), lambda i: (0, 0)),
                  pl.BlockSpec((1, HID), lambda i: (0, 0)),
                  pl.BlockSpec((T, F, F), lambda i: (0, 0, 0)),
                  pl.BlockSpec((T, F), lambda i: (0, 0))],
        out_specs=pl.BlockSpec((HID, EB2), lambda i: (0, i)),
        out_shape=jax.ShapeDtypeStruct((HID, E), jnp.float32),
    )(ga, gb, prot_edge_attr, cw, cbias, pre_W2, pre_b2)

    sT, sqT, mnT, mxT, cnt = pl.kernel(
        _sc_segment_body,
        out_type=[jax.ShapeDtypeStruct((HID * NP,), jnp.float32),
                  jax.ShapeDtypeStruct((HID * NP,), jnp.float32),
                  jax.ShapeDtypeStruct((HID * NP,), jnp.float32),
                  jax.ShapeDtypeStruct((HID * NP,), jnp.float32),
                  jax.ShapeDtypeStruct((NP,), jnp.float32)],
        mesh=plsc.VectorSubcoreMesh(core_axis_name="c", subcore_axis_name="s"),
        scratch_types=[pltpu.VMEM((CPW * NP,), jnp.float32),
                       pltpu.VMEM((CPW * NP,), jnp.float32),
                       pltpu.VMEM((CPW * NP,), jnp.float32),
                       pltpu.VMEM((CPW * NP,), jnp.float32),
                       pltpu.VMEM((NP,), jnp.float32),
                       pltpu.VMEM((CPW * EB,), jnp.float32),
                       pltpu.VMEM((EB,), jnp.int32),
                       pltpu.VMEM((NP,), jnp.int32)],
        compiler_params=pltpu.CompilerParams(needs_layout_passes=False),
    )(mT.reshape(HID * E), dst)
    sT = sT.reshape(HID, NP)
    sqT = sqT.reshape(HID, NP)
    mnT = mnT.reshape(HID, NP)
    mxT = mxT.reshape(HID, NP)

    xp = jnp.concatenate([x, jnp.zeros((NP - N, HID), jnp.float32)], axis=0)
    out = pl.pallas_call(
        _post_kernel,
        grid=(NP // NB,),
        in_specs=[pl.BlockSpec((NB, HID), lambda i: (i, 0)),
                  pl.BlockSpec((HID, NB), lambda i: (0, i)),
                  pl.BlockSpec((HID, NB), lambda i: (0, i)),
                  pl.BlockSpec((HID, NB), lambda i: (0, i)),
                  pl.BlockSpec((HID, NB), lambda i: (0, i)),
                  pl.BlockSpec((1, NB), lambda i: (0, i)),
                  pl.BlockSpec((T, 16 * F, F), lambda i: (0, 0, 0)),
                  pl.BlockSpec((T, F), lambda i: (0, 0)),
                  pl.BlockSpec((T, F, F), lambda i: (0, 0, 0)),
                  pl.BlockSpec((T, F), lambda i: (0, 0)),
                  pl.BlockSpec((HID, HID), lambda i: (0, 0)),
                  pl.BlockSpec((1, HID), lambda i: (0, 0)),
                  pl.BlockSpec((1, HID), lambda i: (0, 0)),
                  pl.BlockSpec((1, HID), lambda i: (0, 0))],
        out_specs=pl.BlockSpec((NB, HID), lambda i: (i, 0)),
        out_shape=jax.ShapeDtypeStruct((NP, HID), jnp.float32),
    )(xp, sT, mnT, mxT, sqT, cnt.reshape(1, NP), post_W1, post_b1, post_W2,
      post_b2, lin_W, lin_b.reshape(1, HID), ln_g.reshape(1, HID),
      ln_b.reshape(1, HID))
    return out[:N]


# speedup vs baseline: 1.1250x; 1.0405x over previous
import jax
import jax.numpy as jnp
import numpy as np
from jax import lax
from jax.experimental import pallas as pl
from jax.experimental.pallas import tpu as pltpu
from jax.experimental.pallas import tpu_sc as plsc

N = 10000
E = 160000
T = 4
F = 64
HID = 256
EDGE_DIM = 16

_DEG_HIST = np.array([0,1,2,5,11,23,44,79,135,216,324,457,605,753,880,966,997,966,880,753,605,457,324,216,135,79,44,23,11,5,2,1,0], dtype=np.float64)
_bins_ = np.arange(_DEG_HIST.shape[0], dtype=np.float64)
_AVG_LOG = float((np.log(_bins_ + 1.0) * _DEG_HIST).sum() / float(_DEG_HIST.sum()))

NP = 10240
NB = 1024
EB2 = 1280
EB = 6400

_NC = 2
_NS = 16
_NW = _NC * _NS
EPW = E // _NW
IC = 1000
GC = 40
CPW = 2
ROUNDS = HID // (_NW * CPW)
BIG = 3.0e38


def _fold_kernel(edge_W_ref, edge_b_ref, pre_W1_ref, pre_b1_ref, cw_ref, cbias_ref):
    for t in range(T):
        w1c = pre_W1_ref[t, 128:192, :]
        cw_ref[:, t * F:(t + 1) * F] = jnp.dot(edge_W_ref[...], w1c,
                                               preferred_element_type=jnp.float32)
        cb = jnp.dot(edge_b_ref[...], w1c, preferred_element_type=jnp.float32)
        cbias_ref[0, t * F:(t + 1) * F] = cb + pre_b1_ref[t, :]


def _tables_kernel(x_ref, pre_W1_ref, a_ref, b_ref):
    for t in range(T):
        xt = x_ref[:, t * F:(t + 1) * F]
        a_ref[:, t * F:(t + 1) * F] = jnp.dot(xt, pre_W1_ref[t, 0:F, :],
                                              preferred_element_type=jnp.float32)
        b_ref[:, t * F:(t + 1) * F] = jnp.dot(xt, pre_W1_ref[t, F:2 * F, :],
                                              preferred_element_type=jnp.float32)


def _sc_gather_body(a_hbm, b_hbm, dst_hbm, src_hbm, ga_hbm, gb_hbm,
                    idx_d, idx_s, rows_a, rows_b, sema, semb):
    wid = lax.axis_index("s") * _NC + lax.axis_index("c")
    base = wid * EPW

    def outer(jj, _):
        off = base + jj * IC
        pltpu.sync_copy(dst_hbm.at[pl.ds(off, IC)], idx_d)
        pltpu.sync_copy(src_hbm.at[pl.ds(off, IC)], idx_s)

        def inner(j, _):
            ca = pltpu.async_copy(a_hbm.at[idx_d.at[pl.ds(j * GC, GC)]],
                                  rows_a, sema)
            cb = pltpu.async_copy(b_hbm.at[idx_s.at[pl.ds(j * GC, GC)]],
                                  rows_b, semb)
            ca.wait()
            cb.wait()
            wa = pltpu.async_copy(rows_a, ga_hbm.at[pl.ds(off + j * GC, GC)],
                                  sema)
            wb = pltpu.async_copy(rows_b, gb_hbm.at[pl.ds(off + j * GC, GC)],
                                  semb)
            wa.wait()
            wb.wait()
            return 0

        lax.fori_loop(0, IC // GC, inner, 0)
        return 0

    lax.fori_loop(0, EPW // IC, outer, 0)


def _edge_kernel(ga_ref, gb_ref, ea_ref, cw_ref, cbias_ref, w2_ref, b2_ref, mT_ref):
    pre = (ga_ref[...] + gb_ref[...]
           + jnp.dot(ea_ref[...], cw_ref[...], preferred_element_type=jnp.float32)
           + cbias_ref[...])
    for t in range(T):
        mt = jnp.maximum(pre[:, t * F:(t + 1) * F], 0.0)
        mT_ref[t * F:(t + 1) * F, :] = (
            lax.dot_general(w2_ref[t], mt, (((0,), (1,)), ((), ())),
                            preferred_element_type=jnp.float32)
            + b2_ref[t, :][:, None])


def _vec_fill(ref, n, valvec):
    def body(p, _):
        ref[pl.ds(p * 16, 16)] = valvec
        return 0
    lax.fori_loop(0, n // 16, body, 0)


def _scatter_minmax(acc, idx, val, is_min):
    def cond(go):
        return go

    def body(_):
        old = plsc.load_gather(acc, [idx])
        need = (val < old) if is_min else (val > old)
        plsc.store_scatter(acc, [idx], val, mask=need)
        return jnp.any(need)

    lax.while_loop(cond, body, jnp.bool_(True))


def _sc_segment_body(mflat_hbm, dst_hbm, sflat_hbm, sqflat_hbm, mnflat_hbm,
                     mxflat_hbm, cnt_hbm,
                     acc_s, acc_q, acc_mn, acc_mx, acc_c, mbuf, dbuf, tmpi,
                     semc):
    wid = lax.axis_index("s") * _NC + lax.axis_index("c")
    zero16 = jnp.zeros((16,), jnp.float32)
    big16 = jnp.full((16,), BIG, jnp.float32)
    one16 = jnp.ones((16,), jnp.float32)

    for r in range(ROUNDS):
        c0 = r * (_NW * CPW) + wid * CPW
        _vec_fill(acc_s, CPW * NP, zero16)
        _vec_fill(acc_q, CPW * NP, zero16)
        _vec_fill(acc_mn, CPW * NP, big16)
        _vec_fill(acc_mx, CPW * NP, -big16)
        if r == 0:
            @pl.when(wid == 0)
            def _():
                def cbody(p, _):
                    acc_c[pl.ds(p * 16, 16)] = zero16
                    return 0
                lax.fori_loop(0, NP // 16, cbody, 0)

        def chunk(ch, _):
            off = ch * EB
            cds = [pltpu.async_copy(dst_hbm.at[pl.ds(off, EB)], dbuf, semc)]
            for k in range(CPW):
                cds.append(pltpu.async_copy(
                    mflat_hbm.at[pl.ds((c0 + k) * E + off, EB)],
                    mbuf.at[pl.ds(k * EB, EB)], semc))
            for cd in cds:
                cd.wait()

            lane = lax.iota(jnp.int32, 16)

            def one_vec(v):
                idx = dbuf[pl.ds(v * 16, 16)]
                plsc.store_scatter(tmpi, [idx], lane)
                back = plsc.load_gather(tmpi, [idx])
                nodup = jnp.all(back == lane)
                vals = [mbuf[pl.ds(k * EB + v * 16, 16)] for k in range(CPW)]
                idxs = [idx + jnp.int32(k * NP) for k in range(CPW)]
                for k in range(CPW):
                    plsc.addupdate_scatter(acc_s, [idxs[k]], vals[k])
                    plsc.addupdate_scatter(acc_q, [idxs[k]], vals[k] * vals[k])

                @pl.when(nodup)
                def _():
                    for k in range(CPW):
                        oldn = plsc.load_gather(acc_mn, [idxs[k]])
                        plsc.store_scatter(acc_mn, [idxs[k]],
                                           jnp.minimum(oldn, vals[k]))
                        oldx = plsc.load_gather(acc_mx, [idxs[k]])
                        plsc.store_scatter(acc_mx, [idxs[k]],
                                           jnp.maximum(oldx, vals[k]))

                @pl.when(jnp.logical_not(nodup))
                def _():
                    for k in range(CPW):
                        _scatter_minmax(acc_mn, idxs[k], vals[k], True)
                        _scatter_minmax(acc_mx, idxs[k], vals[k], False)
                if r == 0:
                    @pl.when(wid == 0)
                    def _():
                        plsc.addupdate_scatter(acc_c, [idx], one16)

            def vec(v, _):
                one_vec(2 * v)
                one_vec(2 * v + 1)
                return 0

            lax.fori_loop(0, EB // 32, vec, 0)
            return 0

        lax.fori_loop(0, E // EB, chunk, 0)

        for k in range(CPW):
            pltpu.sync_copy(acc_s.at[pl.ds(k * NP, NP)],
                            sflat_hbm.at[pl.ds((c0 + k) * NP, NP)])
            pltpu.sync_copy(acc_q.at[pl.ds(k * NP, NP)],
                            sqflat_hbm.at[pl.ds((c0 + k) * NP, NP)])
            pltpu.sync_copy(acc_mn.at[pl.ds(k * NP, NP)],
                            mnflat_hbm.at[pl.ds((c0 + k) * NP, NP)])
            pltpu.sync_copy(acc_mx.at[pl.ds(k * NP, NP)],
                            mxflat_hbm.at[pl.ds((c0 + k) * NP, NP)])
        if r == 0:
            @pl.when(wid == 0)
            def _():
                pltpu.sync_copy(acc_c, cnt_hbm)


def _post_kernel(x_ref, sT_ref, mnT_ref, mxT_ref, sqT_ref, cnt_ref,
                 pw1_ref, pb1_ref, pw2_ref, pb2_ref, lw_ref, lb_ref,
                 lg_ref, lbeta_ref, out_ref):
    cnt = cnt_ref[...]
    deg = jnp.maximum(cnt, 1.0)
    inv_deg = 1.0 / deg
    has = (cnt > 0.0).astype(jnp.float32)
    lg1d = jnp.log(deg + 1.0)
    amp = lg1d * (1.0 / _AVG_LOG)
    att = _AVG_LOG / lg1d

    s = sT_ref[...]
    mean = s * inv_deg
    mn = mnT_ref[...] * has
    mx = mxT_ref[...] * has
    msq = sqT_ref[...] * inv_deg
    var = jnp.maximum(msq - mean * mean, 0.0)
    std = jnp.sqrt(var + 1e-5)

    dn = (((0,), (0,)), ((), ()))
    o_parts = []
    for t in range(T):
        sl = slice(t * F, (t + 1) * F)
        aggT = jnp.concatenate([s[sl], mean[sl], mn[sl], mx[sl], std[sl]],
                               axis=0)
        w = pw1_ref[t]
        otT = lax.dot_general(w[0:F], x_ref[:, sl], (((0,), (1,)), ((), ())),
                              preferred_element_type=jnp.float32)
        otT = otT + lax.dot_general(w[F:F + 5 * F], aggT, dn,
                                    preferred_element_type=jnp.float32)
        otT = otT + amp * lax.dot_general(w[F + 5 * F:F + 10 * F], aggT, dn,
                                          preferred_element_type=jnp.float32)
        otT = otT + att * lax.dot_general(w[F + 10 * F:F + 15 * F], aggT, dn,
                                          preferred_element_type=jnp.float32)
        otT = jnp.maximum(otT + pb1_ref[t, :][:, None], 0.0)
        otT = (lax.dot_general(pw2_ref[t], otT, dn,
                               preferred_element_type=jnp.float32)
               + pb2_ref[t, :][:, None])
        o_parts.append(otT)
    oT = jnp.concatenate(o_parts, axis=0)

    o = lax.dot_general(oT, lw_ref[...], dn,
                        preferred_element_type=jnp.float32) + lb_ref[0, :][None, :]
    mu = jnp.mean(o, axis=1, keepdims=True)
    vv = jnp.mean((o - mu) ** 2, axis=1, keepdims=True)
    ln = (o - mu) * lax.rsqrt(vv + 1e-5) * lg_ref[0, :][None, :] + lbeta_ref[0, :][None, :]
    out_ref[...] = x_ref[...] + jnp.maximum(ln, 0.0)


def kernel(x, prot_edge_index, prot_edge_attr, edge_W, edge_b, pre_W1, pre_b1,
           pre_W2, pre_b2, post_W1, post_b1, post_W2, post_b2, lin_W, lin_b,
           ln_g, ln_b):
    src = prot_edge_index[0]
    dst = prot_edge_index[1]

    cw, cbias = pl.pallas_call(
        _fold_kernel,
        out_shape=[jax.ShapeDtypeStruct((EDGE_DIM, HID), jnp.float32),
                   jax.ShapeDtypeStruct((1, HID), jnp.float32)],
    )(edge_W, edge_b, pre_W1, pre_b1)

    a_tab, b_tab = pl.pallas_call(
        _tables_kernel,
        grid=(N // 1000,),
        in_specs=[pl.BlockSpec((1000, HID), lambda i: (i, 0)),
                  pl.BlockSpec((T, 3 * F, F), lambda i: (0, 0, 0))],
        out_specs=[pl.BlockSpec((1000, HID), lambda i: (i, 0)),
                   pl.BlockSpec((1000, HID), lambda i: (i, 0))],
        out_shape=[jax.ShapeDtypeStruct((N, HID), jnp.float32),
                   jax.ShapeDtypeStruct((N, HID), jnp.float32)],
    )(x, pre_W1)

    mesh = plsc.VectorSubcoreMesh(core_axis_name="c", subcore_axis_name="s")
    ga, gb = pl.kernel(
        _sc_gather_body,
        out_type=[jax.ShapeDtypeStruct((E, HID), jnp.float32),
                  jax.ShapeDtypeStruct((E, HID), jnp.float32)],
        mesh=mesh,
        scratch_types=[pltpu.VMEM((IC,), jnp.int32),
                       pltpu.VMEM((IC,), jnp.int32),
                       pltpu.VMEM((GC, HID), jnp.float32),
                       pltpu.VMEM((GC, HID), jnp.float32),
                       pltpu.SemaphoreType.DMA,
                       pltpu.SemaphoreType.DMA],
    )(a_tab, b_tab, dst, src)

    mT = pl.pallas_call(
        _edge_kernel,
        grid=(E // EB2,),
        in_specs=[pl.BlockSpec((EB2, HID), lambda i: (i, 0)),
                  pl.BlockSpec((EB2, HID), lambda i: (i, 0)),
                  pl.BlockSpec((EB2, EDGE_DIM), lambda i: (i, 0)),
                  pl.BlockSpec((EDGE_DIM, HID), lambda i: (0, 0)),
                  pl.BlockSpec((1, HID), lambda i: (0, 0)),
                  pl.BlockSpec((T, F, F), lambda i: (0, 0, 0)),
                  pl.BlockSpec((T, F), lambda i: (0, 0))],
        out_specs=pl.BlockSpec((HID, EB2), lambda i: (0, i)),
        out_shape=jax.ShapeDtypeStruct((HID, E), jnp.float32),
    )(ga, gb, prot_edge_attr, cw, cbias, pre_W2, pre_b2)

    sT, sqT, mnT, mxT, cnt = pl.kernel(
        _sc_segment_body,
        out_type=[jax.ShapeDtypeStruct((HID * NP,), jnp.float32),
                  jax.ShapeDtypeStruct((HID * NP,), jnp.float32),
                  jax.ShapeDtypeStruct((HID * NP,), jnp.float32),
                  jax.ShapeDtypeStruct((HID * NP,), jnp.float32),
                  jax.ShapeDtypeStruct((NP,), jnp.float32)],
        mesh=plsc.VectorSubcoreMesh(core_axis_name="c", subcore_axis_name="s"),
        scratch_types=[pltpu.VMEM((CPW * NP,), jnp.float32),
                       pltpu.VMEM((CPW * NP,), jnp.float32),
                       pltpu.VMEM((CPW * NP,), jnp.float32),
                       pltpu.VMEM((CPW * NP,), jnp.float32),
                       pltpu.VMEM((NP,), jnp.float32),
                       pltpu.VMEM((CPW * EB,), jnp.float32),
                       pltpu.VMEM((EB,), jnp.int32),
                       pltpu.VMEM((NP,), jnp.int32),
                       pltpu.SemaphoreType.DMA],
        compiler_params=pltpu.CompilerParams(needs_layout_passes=False),
    )(mT.reshape(HID * E), dst)
    sT = sT.reshape(HID, NP)
    sqT = sqT.reshape(HID, NP)
    mnT = mnT.reshape(HID, NP)
    mxT = mxT.reshape(HID, NP)

    xp = jnp.concatenate([x, jnp.zeros((NP - N, HID), jnp.float32)], axis=0)
    out = pl.pallas_call(
        _post_kernel,
        grid=(NP // NB,),
        in_specs=[pl.BlockSpec((NB, HID), lambda i: (i, 0)),
                  pl.BlockSpec((HID, NB), lambda i: (0, i)),
                  pl.BlockSpec((HID, NB), lambda i: (0, i)),
                  pl.BlockSpec((HID, NB), lambda i: (0, i)),
                  pl.BlockSpec((HID, NB), lambda i: (0, i)),
                  pl.BlockSpec((1, NB), lambda i: (0, i)),
                  pl.BlockSpec((T, 16 * F, F), lambda i: (0, 0, 0)),
                  pl.BlockSpec((T, F), lambda i: (0, 0)),
                  pl.BlockSpec((T, F, F), lambda i: (0, 0, 0)),
                  pl.BlockSpec((T, F), lambda i: (0, 0)),
                  pl.BlockSpec((HID, HID), lambda i: (0, 0)),
                  pl.BlockSpec((1, HID), lambda i: (0, 0)),
                  pl.BlockSpec((1, HID), lambda i: (0, 0)),
                  pl.BlockSpec((1, HID), lambda i: (0, 0))],
        out_specs=pl.BlockSpec((NB, HID), lambda i: (i, 0)),
        out_shape=jax.ShapeDtypeStruct((NP, HID), jnp.float32),
    )(xp, sT, mnT, mxT, sqT, cnt.reshape(1, NP), post_W1, post_b1, post_W2,
      post_b2, lin_W, lin_b.reshape(1, HID), ln_g.reshape(1, HID),
      ln_b.reshape(1, HID))
    return out[:N]


# speedup vs baseline: 1.1384x; 1.0119x over previous
import jax
import jax.numpy as jnp
import numpy as np
from jax import lax
from jax.experimental import pallas as pl
from jax.experimental.pallas import tpu as pltpu
from jax.experimental.pallas import tpu_sc as plsc

N = 10000
E = 160000
T = 4
F = 64
HID = 256
EDGE_DIM = 16

_DEG_HIST = np.array([0,1,2,5,11,23,44,79,135,216,324,457,605,753,880,966,997,966,880,753,605,457,324,216,135,79,44,23,11,5,2,1,0], dtype=np.float64)
_bins_ = np.arange(_DEG_HIST.shape[0], dtype=np.float64)
_AVG_LOG = float((np.log(_bins_ + 1.0) * _DEG_HIST).sum() / float(_DEG_HIST.sum()))

NP = 10240
NB = 1024
EB2 = 1280
EB = 6400

_NC = 2
_NS = 16
_NW = _NC * _NS
EPW = E // _NW
IC = 1000
GC = 40
CPW = 2
ROUNDS = HID // (_NW * CPW)
BIG = 3.0e38


def _fold_kernel(edge_W_ref, edge_b_ref, pre_W1_ref, pre_b1_ref, cw_ref, cbias_ref):
    for t in range(T):
        w1c = pre_W1_ref[t, 128:192, :]
        cw_ref[:, t * F:(t + 1) * F] = jnp.dot(edge_W_ref[...], w1c,
                                               preferred_element_type=jnp.float32)
        cb = jnp.dot(edge_b_ref[...], w1c, preferred_element_type=jnp.float32)
        cbias_ref[0, t * F:(t + 1) * F] = cb + pre_b1_ref[t, :]


def _tables_kernel(x_ref, pre_W1_ref, a_ref, b_ref):
    for t in range(T):
        xt = x_ref[:, t * F:(t + 1) * F]
        a_ref[:, t * F:(t + 1) * F] = jnp.dot(xt, pre_W1_ref[t, 0:F, :],
                                              preferred_element_type=jnp.float32)
        b_ref[:, t * F:(t + 1) * F] = jnp.dot(xt, pre_W1_ref[t, F:2 * F, :],
                                              preferred_element_type=jnp.float32)


def _sc_gather_body(a_hbm, b_hbm, dst_hbm, src_hbm, ga_hbm, gb_hbm,
                    idx_d, idx_s, rows_a, rows_b, rows_a2, rows_b2,
                    sema, semb, sema2, semb2):
    wid = lax.axis_index("s") * _NC + lax.axis_index("c")
    base = wid * EPW

    def outer(jj, _):
        off = base + jj * IC
        pltpu.sync_copy(dst_hbm.at[pl.ds(off, IC)], idx_d)
        pltpu.sync_copy(src_hbm.at[pl.ds(off, IC)], idx_s)

        def gathers(c, ra, rb, s1, s2):
            ca = pltpu.async_copy(a_hbm.at[idx_d.at[pl.ds(c * GC, GC)]],
                                  ra, s1)
            cb = pltpu.async_copy(b_hbm.at[idx_s.at[pl.ds(c * GC, GC)]],
                                  rb, s2)
            return ca, cb

        def writes(c, ra, rb, s1, s2):
            wa = pltpu.async_copy(ra, ga_hbm.at[pl.ds(off + c * GC, GC)], s1)
            wb = pltpu.async_copy(rb, gb_hbm.at[pl.ds(off + c * GC, GC)], s2)
            return wa, wb

        def inner(j, _):
            c1 = 2 * j
            c2 = 2 * j + 1
            ca1, cb1 = gathers(c1, rows_a, rows_b, sema, semb)
            ca2, cb2 = gathers(c2, rows_a2, rows_b2, sema2, semb2)
            ca1.wait()
            cb1.wait()
            wa1, wb1 = writes(c1, rows_a, rows_b, sema, semb)
            ca2.wait()
            cb2.wait()
            wa2, wb2 = writes(c2, rows_a2, rows_b2, sema2, semb2)
            wa1.wait()
            wb1.wait()
            wa2.wait()
            wb2.wait()
            return 0

        npairs = (IC // GC) // 2
        lax.fori_loop(0, npairs, inner, 0)
        ct = IC // GC - 1
        ca, cb = gathers(ct, rows_a, rows_b, sema, semb)
        ca.wait()
        cb.wait()
        wa, wb = writes(ct, rows_a, rows_b, sema, semb)
        wa.wait()
        wb.wait()
        return 0

    lax.fori_loop(0, EPW // IC, outer, 0)


def _edge_kernel(ga_ref, gb_ref, ea_ref, cw_ref, cbias_ref, w2_ref, b2_ref, mT_ref):
    pre = (ga_ref[...] + gb_ref[...]
           + jnp.dot(ea_ref[...], cw_ref[...], preferred_element_type=jnp.float32)
           + cbias_ref[...])
    for t in range(T):
        mt = jnp.maximum(pre[:, t * F:(t + 1) * F], 0.0)
        mT_ref[t * F:(t + 1) * F, :] = (
            lax.dot_general(w2_ref[t], mt, (((0,), (1,)), ((), ())),
                            preferred_element_type=jnp.float32)
            + b2_ref[t, :][:, None])


def _vec_fill(ref, n, valvec):
    def body(p, _):
        ref[pl.ds(p * 16, 16)] = valvec
        return 0
    lax.fori_loop(0, n // 16, body, 0)


def _scatter_minmax(acc, idx, val, is_min):
    def cond(go):
        return go

    def body(_):
        old = plsc.load_gather(acc, [idx])
        need = (val < old) if is_min else (val > old)
        plsc.store_scatter(acc, [idx], val, mask=need)
        return jnp.any(need)

    lax.while_loop(cond, body, jnp.bool_(True))


def _sc_segment_body(mflat_hbm, dst_hbm, sflat_hbm, sqflat_hbm, mnflat_hbm,
                     mxflat_hbm, cnt_hbm,
                     acc_s, acc_q, acc_mn, acc_mx, acc_c, mbuf, dbuf, tmpi,
                     semc):
    wid = lax.axis_index("s") * _NC + lax.axis_index("c")
    zero16 = jnp.zeros((16,), jnp.float32)
    big16 = jnp.full((16,), BIG, jnp.float32)
    one16 = jnp.ones((16,), jnp.float32)

    for r in range(ROUNDS):
        c0 = r * (_NW * CPW) + wid * CPW
        _vec_fill(acc_s, CPW * NP, zero16)
        _vec_fill(acc_q, CPW * NP, zero16)
        _vec_fill(acc_mn, CPW * NP, big16)
        _vec_fill(acc_mx, CPW * NP, -big16)
        if r == 0:
            @pl.when(wid == 0)
            def _():
                def cbody(p, _):
                    acc_c[pl.ds(p * 16, 16)] = zero16
                    return 0
                lax.fori_loop(0, NP // 16, cbody, 0)

        def chunk(ch, _):
            off = ch * EB
            cds = [pltpu.async_copy(dst_hbm.at[pl.ds(off, EB)], dbuf, semc)]
            for k in range(CPW):
                cds.append(pltpu.async_copy(
                    mflat_hbm.at[pl.ds((c0 + k) * E + off, EB)],
                    mbuf.at[pl.ds(k * EB, EB)], semc))
            for cd in cds:
                cd.wait()

            lane = lax.iota(jnp.int32, 16)

            def one_vec(v):
                idx = dbuf[pl.ds(v * 16, 16)]
                plsc.store_scatter(tmpi, [idx], lane)
                back = plsc.load_gather(tmpi, [idx])
                nodup = jnp.all(back == lane)
                vals = [mbuf[pl.ds(k * EB + v * 16, 16)] for k in range(CPW)]
                idxs = [idx + jnp.int32(k * NP) for k in range(CPW)]
                for k in range(CPW):
                    plsc.addupdate_scatter(acc_s, [idxs[k]], vals[k])
                    plsc.addupdate_scatter(acc_q, [idxs[k]], vals[k] * vals[k])

                @pl.when(nodup)
                def _():
                    for k in range(CPW):
                        oldn = plsc.load_gather(acc_mn, [idxs[k]])
                        plsc.store_scatter(acc_mn, [idxs[k]],
                                           jnp.minimum(oldn, vals[k]))
                        oldx = plsc.load_gather(acc_mx, [idxs[k]])
                        plsc.store_scatter(acc_mx, [idxs[k]],
                                           jnp.maximum(oldx, vals[k]))

                @pl.when(jnp.logical_not(nodup))
                def _():
                    for k in range(CPW):
                        _scatter_minmax(acc_mn, idxs[k], vals[k], True)
                        _scatter_minmax(acc_mx, idxs[k], vals[k], False)
                if r == 0:
                    @pl.when(wid == 0)
                    def _():
                        plsc.addupdate_scatter(acc_c, [idx], one16)

            def vec(v, _):
                one_vec(2 * v)
                one_vec(2 * v + 1)
                return 0

            lax.fori_loop(0, EB // 32, vec, 0)
            return 0

        lax.fori_loop(0, E // EB, chunk, 0)

        for k in range(CPW):
            pltpu.sync_copy(acc_s.at[pl.ds(k * NP, NP)],
                            sflat_hbm.at[pl.ds((c0 + k) * NP, NP)])
            pltpu.sync_copy(acc_q.at[pl.ds(k * NP, NP)],
                            sqflat_hbm.at[pl.ds((c0 + k) * NP, NP)])
            pltpu.sync_copy(acc_mn.at[pl.ds(k * NP, NP)],
                            mnflat_hbm.at[pl.ds((c0 + k) * NP, NP)])
            pltpu.sync_copy(acc_mx.at[pl.ds(k * NP, NP)],
                            mxflat_hbm.at[pl.ds((c0 + k) * NP, NP)])
        if r == 0:
            @pl.when(wid == 0)
            def _():
                pltpu.sync_copy(acc_c, cnt_hbm)


def _post_kernel(x_ref, sT_ref, mnT_ref, mxT_ref, sqT_ref, cnt_ref,
                 pw1_ref, pb1_ref, pw2_ref, pb2_ref, lw_ref, lb_ref,
                 lg_ref, lbeta_ref, out_ref):
    cnt = cnt_ref[...]
    deg = jnp.maximum(cnt, 1.0)
    inv_deg = 1.0 / deg
    has = (cnt > 0.0).astype(jnp.float32)
    lg1d = jnp.log(deg + 1.0)
    amp = lg1d * (1.0 / _AVG_LOG)
    att = _AVG_LOG / lg1d

    s = sT_ref[...]
    mean = s * inv_deg
    mn = mnT_ref[...] * has
    mx = mxT_ref[...] * has
    msq = sqT_ref[...] * inv_deg
    var = jnp.maximum(msq - mean * mean, 0.0)
    std = jnp.sqrt(var + 1e-5)

    dn = (((0,), (0,)), ((), ()))
    o_parts = []
    for t in range(T):
        sl = slice(t * F, (t + 1) * F)
        aggT = jnp.concatenate([s[sl], mean[sl], mn[sl], mx[sl], std[sl]],
                               axis=0)
        w = pw1_ref[t]
        otT = lax.dot_general(w[0:F], x_ref[:, sl], (((0,), (1,)), ((), ())),
                              preferred_element_type=jnp.float32)
        otT = otT + lax.dot_general(w[F:F + 5 * F], aggT, dn,
                                    preferred_element_type=jnp.float32)
        otT = otT + amp * lax.dot_general(w[F + 5 * F:F + 10 * F], aggT, dn,
                                          preferred_element_type=jnp.float32)
        otT = otT + att * lax.dot_general(w[F + 10 * F:F + 15 * F], aggT, dn,
                                          preferred_element_type=jnp.float32)
        otT = jnp.maximum(otT + pb1_ref[t, :][:, None], 0.0)
        otT = (lax.dot_general(pw2_ref[t], otT, dn,
                               preferred_element_type=jnp.float32)
               + pb2_ref[t, :][:, None])
        o_parts.append(otT)
    oT = jnp.concatenate(o_parts, axis=0)

    o = lax.dot_general(oT, lw_ref[...], dn,
                        preferred_element_type=jnp.float32) + lb_ref[0, :][None, :]
    mu = jnp.mean(o, axis=1, keepdims=True)
    vv = jnp.mean((o - mu) ** 2, axis=1, keepdims=True)
    ln = (o - mu) * lax.rsqrt(vv + 1e-5) * lg_ref[0, :][None, :] + lbeta_ref[0, :][None, :]
    out_ref[...] = x_ref[...] + jnp.maximum(ln, 0.0)


def kernel(x, prot_edge_index, prot_edge_attr, edge_W, edge_b, pre_W1, pre_b1,
           pre_W2, pre_b2, post_W1, post_b1, post_W2, post_b2, lin_W, lin_b,
           ln_g, ln_b):
    src = prot_edge_index[0]
    dst = prot_edge_index[1]

    cw, cbias = pl.pallas_call(
        _fold_kernel,
        out_shape=[jax.ShapeDtypeStruct((EDGE_DIM, HID), jnp.float32),
                   jax.ShapeDtypeStruct((1, HID), jnp.float32)],
    )(edge_W, edge_b, pre_W1, pre_b1)

    a_tab, b_tab = pl.pallas_call(
        _tables_kernel,
        grid=(N // 1000,),
        in_specs=[pl.BlockSpec((1000, HID), lambda i: (i, 0)),
                  pl.BlockSpec((T, 3 * F, F), lambda i: (0, 0, 0))],
        out_specs=[pl.BlockSpec((1000, HID), lambda i: (i, 0)),
                   pl.BlockSpec((1000, HID), lambda i: (i, 0))],
        out_shape=[jax.ShapeDtypeStruct((N, HID), jnp.float32),
                   jax.ShapeDtypeStruct((N, HID), jnp.float32)],
    )(x, pre_W1)

    mesh = plsc.VectorSubcoreMesh(core_axis_name="c", subcore_axis_name="s")
    ga, gb = pl.kernel(
        _sc_gather_body,
        out_type=[jax.ShapeDtypeStruct((E, HID), jnp.float32),
                  jax.ShapeDtypeStruct((E, HID), jnp.float32)],
        mesh=mesh,
        scratch_types=[pltpu.VMEM((IC,), jnp.int32),
                       pltpu.VMEM((IC,), jnp.int32),
                       pltpu.VMEM((GC, HID), jnp.float32),
                       pltpu.VMEM((GC, HID), jnp.float32),
                       pltpu.VMEM((GC, HID), jnp.float32),
                       pltpu.VMEM((GC, HID), jnp.float32),
                       pltpu.SemaphoreType.DMA,
                       pltpu.SemaphoreType.DMA,
                       pltpu.SemaphoreType.DMA,
                       pltpu.SemaphoreType.DMA],
    )(a_tab, b_tab, dst, src)

    mT = pl.pallas_call(
        _edge_kernel,
        grid=(E // EB2,),
        in_specs=[pl.BlockSpec((EB2, HID), lambda i: (i, 0)),
                  pl.BlockSpec((EB2, HID), lambda i: (i, 0)),
                  pl.BlockSpec((EB2, EDGE_DIM), lambda i: (i, 0)),
                  pl.BlockSpec((EDGE_DIM, HID), lambda i: (0, 0)),
                  pl.BlockSpec((1, HID), lambda i: (0, 0)),
                  pl.BlockSpec((T, F, F), lambda i: (0, 0, 0)),
                  pl.BlockSpec((T, F), lambda i: (0, 0))],
        out_specs=pl.BlockSpec((HID, EB2), lambda i: (0, i)),
        out_shape=jax.ShapeDtypeStruct((HID, E), jnp.float32),
    )(ga, gb, prot_edge_attr, cw, cbias, pre_W2, pre_b2)

    sT, sqT, mnT, mxT, cnt = pl.kernel(
        _sc_segment_body,
        out_type=[jax.ShapeDtypeStruct((HID * NP,), jnp.float32),
                  jax.ShapeDtypeStruct((HID * NP,), jnp.float32),
                  jax.ShapeDtypeStruct((HID * NP,), jnp.float32),
                  jax.ShapeDtypeStruct((HID * NP,), jnp.float32),
                  jax.ShapeDtypeStruct((NP,), jnp.float32)],
        mesh=plsc.VectorSubcoreMesh(core_axis_name="c", subcore_axis_name="s"),
        scratch_types=[pltpu.VMEM((CPW * NP,), jnp.float32),
                       pltpu.VMEM((CPW * NP,), jnp.float32),
                       pltpu.VMEM((CPW * NP,), jnp.float32),
                       pltpu.VMEM((CPW * NP,), jnp.float32),
                       pltpu.VMEM((NP,), jnp.float32),
                       pltpu.VMEM((CPW * EB,), jnp.float32),
                       pltpu.VMEM((EB,), jnp.int32),
                       pltpu.VMEM((NP,), jnp.int32),
                       pltpu.SemaphoreType.DMA],
        compiler_params=pltpu.CompilerParams(needs_layout_passes=False),
    )(mT.reshape(HID * E), dst)
    sT = sT.reshape(HID, NP)
    sqT = sqT.reshape(HID, NP)
    mnT = mnT.reshape(HID, NP)
    mxT = mxT.reshape(HID, NP)

    xp = jnp.concatenate([x, jnp.zeros((NP - N, HID), jnp.float32)], axis=0)
    out = pl.pallas_call(
        _post_kernel,
        grid=(NP // NB,),
        in_specs=[pl.BlockSpec((NB, HID), lambda i: (i, 0)),
                  pl.BlockSpec((HID, NB), lambda i: (0, i)),
                  pl.BlockSpec((HID, NB), lambda i: (0, i)),
                  pl.BlockSpec((HID, NB), lambda i: (0, i)),
                  pl.BlockSpec((HID, NB), lambda i: (0, i)),
                  pl.BlockSpec((1, NB), lambda i: (0, i)),
                  pl.BlockSpec((T, 16 * F, F), lambda i: (0, 0, 0)),
                  pl.BlockSpec((T, F), lambda i: (0, 0)),
                  pl.BlockSpec((T, F, F), lambda i: (0, 0, 0)),
                  pl.BlockSpec((T, F), lambda i: (0, 0)),
                  pl.BlockSpec((HID, HID), lambda i: (0, 0)),
                  pl.BlockSpec((1, HID), lambda i: (0, 0)),
                  pl.BlockSpec((1, HID), lambda i: (0, 0)),
                  pl.BlockSpec((1, HID), lambda i: (0, 0))],
        out_specs=pl.BlockSpec((NB, HID), lambda i: (i, 0)),
        out_shape=jax.ShapeDtypeStruct((NP, HID), jnp.float32),
    )(xp, sT, mnT, mxT, sqT, cnt.reshape(1, NP), post_W1, post_b1, post_W2,
      post_b2, lin_W, lin_b.reshape(1, HID), ln_g.reshape(1, HID),
      ln_b.reshape(1, HID))
    return out[:N]
